# Initial kernel scaffold; baseline (speedup 1.0000x reference)
import jax
import jax.numpy as jnp
from jax.experimental import pallas as pl


def kernel(x_t, x_s, edge_weight_t, edge_weight_s, edge_weight_t1, edge_weight_s1, params, edge_index_t, edge_index_s, edge_index, edge_index_t1, edge_index_s1, edge_index1, pos_t, pos_s, n_batch1, s_batch1):
    raise NotImplementedError("write your pallas kernel here")


# speedup vs baseline: 1.9039x; 1.9039x over previous
import functools

import jax
import jax.numpy as jnp
from jax import lax
from jax.experimental import pallas as pl
from jax.experimental.pallas import tpu as pltpu
from jax.experimental.pallas import tpu_sc as plsc

F32 = jnp.float32
I32 = jnp.int32

_CH = 64
_NW = 32
_STRIDE = _NW * _CH
_ROW_PAD = 1024
_SPMEM_BUDGET = 4 * 1024 * 1024


def _ceil_to(x, m):
    return -(-x // m) * m


def _feat_chunks(d, n_pad):
    max_dc = min(512, (_SPMEM_BUDGET // (4 * n_pad)) // 16 * 16)
    nc = -(-d // max_dc)
    base = (d // nc) // 16 * 16
    sizes = [base] * (nc - 1) + [d - base * (nc - 1)]
    assert all(16 <= s <= max_dc and s % 16 == 0 for s in sizes), (d, n_pad, sizes)
    return tuple(sizes)


@functools.lru_cache(maxsize=None)
def _sc_scatter_builder(nx, dc, nnz_pad, n_pad, mode):
    nchunk = nnz_pad // (_NW * _CH)
    rows_per_tile = n_pad // 16
    n_copies = rows_per_tile // _CH
    ncol = dc // 16
    mesh = plsc.VectorSubcoreMesh(core_axis_name="c", subcore_axis_name="s")

    scratch = []
    if mode != "ones":
        scratch.append(pltpu.VMEM((_CH,), I32))
    scratch.append(pltpu.VMEM((_CH,), I32))
    if mode == "gs":
        scratch.append(pltpu.VMEM((_CH,), F32))
    scratch += [
        pltpu.VMEM((_CH, dc), F32),
        pltpu.VMEM_SHARED((n_pad, dc), F32),
        pltpu.SemaphoreType.DMA,
    ]

    def kern(*args):
        if mode == "gs":
            (x_hbm, gidx_hbm, dst_hbm, w_hbm, out_hbm,
             gidx_v, dst_v, w_v, rows_v, acc, sem) = args
        elif mode == "g":
            (x_hbm, gidx_hbm, dst_hbm, out_hbm,
             gidx_v, dst_v, rows_v, acc, sem) = args
        else:
            (dst_hbm, out_hbm, dst_v, rows_v, acc, sem) = args
        c = lax.axis_index("c")
        s = lax.axis_index("s")
        wid = s * 2 + c
        t0 = s * rows_per_tile

        def fill(val):
            vec = jnp.full((16,), val, F32)

            def row(r, carry):
                for k in range(ncol):
                    rows_v[r, pl.ds(16 * k, 16)] = vec
                return carry

            lax.fori_loop(0, _CH, row, 0)

        fill(0.0)
        for j in range(n_copies):
            pltpu.sync_copy(rows_v, acc.at[pl.ds(t0 + j * _CH, _CH)])
        plsc.subcore_barrier()
        if mode == "ones":
            fill(1.0)

        def chunk(ci, carry):
            base = (wid * nchunk + ci) * _CH
            pltpu.sync_copy(dst_hbm.at[pl.ds(base, _CH)], dst_v)
            if mode != "ones":
                pltpu.sync_copy(gidx_hbm.at[pl.ds(base, _CH)], gidx_v)
                pltpu.async_copy(x_hbm.at[gidx_v], rows_v, sem).wait()
            if mode == "gs":
                pltpu.sync_copy(w_hbm.at[pl.ds(base, _CH)], w_v)

                def sgrp(g, carry2):
                    wg = w_v[pl.ds(g * 16, 16)]
                    for r16 in range(16):
                        ws = wg.at[jnp.full((16,), r16, I32)].get(
                            mode="promise_in_bounds")
                        for k in range(ncol):
                            sl = pl.ds(16 * k, 16)
                            rows_v[g * 16 + r16, sl] = rows_v[g * 16 + r16, sl] * ws
                    return carry2

                lax.fori_loop(0, _CH // 16, sgrp, 0)
            pltpu.sync_copy(rows_v, acc.at[dst_v], add=True)
            return carry

        lax.fori_loop(0, nchunk, chunk, 0)
        plsc.subcore_barrier()
        for j in range(n_copies):
            sl = pl.ds(t0 + j * _CH, _CH)
            pltpu.sync_copy(acc.at[sl], out_hbm.at[c, sl])

    return pl.kernel(
        kern,
        out_type=jax.ShapeDtypeStruct((2, n_pad, dc), F32),
        mesh=mesh,
        scratch_types=scratch,
        compiler_params=pltpu.CompilerParams(use_tc_tiling_on_sc=False),
    )


def _pad_entries(arr, nnz_pad, value):
    n = arr.shape[0]
    if n == nnz_pad:
        return arr
    return jnp.concatenate([arr, jnp.full((nnz_pad - n,), value, arr.dtype)])


def _sc_scatter(x, gidx, dst, w, n_out):
    nnz = dst.shape[0]
    nnz_pad = _ceil_to(nnz, _STRIDE)
    n_pad = _ceil_to(n_out + 1, _ROW_PAD)
    gidx_p = _pad_entries(gidx, nnz_pad, 0)
    dst_p = _pad_entries(dst, nnz_pad, n_out)
    w_p = None if w is None else _pad_entries(w, nnz_pad, 0.0)
    d = x.shape[1]
    parts = []
    c0 = 0
    chunks = _feat_chunks(d, n_pad)
    for dc in chunks:
        xc = lax.slice_in_dim(x, c0, c0 + dc, axis=1)
        if w is None:
            fn = _sc_scatter_builder(x.shape[0], dc, nnz_pad, n_pad, "g")
            parts.append(fn(xc, gidx_p, dst_p))
        else:
            fn = _sc_scatter_builder(x.shape[0], dc, nnz_pad, n_pad, "gs")
            parts.append(fn(xc, gidx_p, dst_p, w_p))
        c0 += dc
    return parts, chunks


def _sc_bincount(idx, n_out):
    nnz = idx.shape[0]
    nnz_pad = _ceil_to(nnz, _STRIDE)
    n_pad = _ceil_to(n_out + 1, _ROW_PAD)
    dst_p = _pad_entries(idx, nnz_pad, n_out)
    fn = _sc_scatter_builder(0, 16, nnz_pad, n_pad, "ones")
    return fn(dst_p)


def _row_block(n_pad, d_tot):
    br = 2048 if n_pad % 2048 == 0 else n_pad
    if d_tot >= 704 and br > 1024:
        br = 1024
    return br


@functools.lru_cache(maxsize=None)
def _mm_stats_builder(n_pad, n_true, k, f):
    br = _row_block(n_pad, k)
    grid = n_pad // br

    def body(x_ref, w_ref, y_ref, st_ref):
        y = jnp.dot(x_ref[...], w_ref[...], preferred_element_type=F32)
        y_ref[...] = y
        i = pl.program_id(0)

        @pl.when(i == 0)
        def _():
            st_ref[...] = jnp.zeros_like(st_ref)

        row = i * br + lax.broadcasted_iota(I32, (br, 1), 0)
        ym = jnp.where(row < n_true, y, 0.0)
        st_ref[0:1, :] = st_ref[0:1, :] + jnp.sum(ym, axis=0, keepdims=True)
        st_ref[1:2, :] = st_ref[1:2, :] + jnp.sum(ym * ym, axis=0, keepdims=True)

    return pl.pallas_call(
        body,
        grid=(grid,),
        in_specs=[pl.BlockSpec((br, k), lambda i: (i, 0)),
                  pl.BlockSpec((k, f), lambda i: (0, 0))],
        out_specs=[pl.BlockSpec((br, f), lambda i: (i, 0)),
                   pl.BlockSpec((8, f), lambda i: (0, 0))],
        out_shape=[jax.ShapeDtypeStruct((n_pad, f), F32),
                   jax.ShapeDtypeStruct((8, f), F32)],
    )


@functools.lru_cache(maxsize=None)
def _laguerre_builder(n_pad, n_true, d, f, chunks):
    br = _row_block(n_pad, d)
    grid = n_pad // br
    nchunks = len(chunks)

    def body(*refs):
        x_ref = refs[0]
        p_refs = refs[1:1 + nchunks]
        w0_ref, w1_ref, y_ref, st_ref = refs[1 + nchunks:]
        x = x_ref[...]
        lx = jnp.concatenate([p[0] + p[1] for p in p_refs], axis=-1) \
            if nchunks > 1 else (p_refs[0][0] + p_refs[0][1])
        y = (jnp.dot(x, w0_ref[...], preferred_element_type=F32)
             + jnp.dot(x - lx, w1_ref[...], preferred_element_type=F32))
        y_ref[...] = y
        i = pl.program_id(0)

        @pl.when(i == 0)
        def _():
            st_ref[...] = jnp.zeros_like(st_ref)

        row = i * br + lax.broadcasted_iota(I32, (br, 1), 0)
        ym = jnp.where(row < n_true, y, 0.0)
        st_ref[0:1, :] = st_ref[0:1, :] + jnp.sum(ym, axis=0, keepdims=True)
        st_ref[1:2, :] = st_ref[1:2, :] + jnp.sum(ym * ym, axis=0, keepdims=True)

    in_specs = [pl.BlockSpec((br, d), lambda i: (i, 0))]
    for dc in chunks:
        in_specs.append(pl.BlockSpec((2, br, dc), lambda i: (0, i, 0)))
    in_specs += [pl.BlockSpec((d, f), lambda i: (0, 0)),
                 pl.BlockSpec((d, f), lambda i: (0, 0))]
    return pl.pallas_call(
        body,
        grid=(grid,),
        in_specs=in_specs,
        out_specs=[pl.BlockSpec((br, f), lambda i: (i, 0)),
                   pl.BlockSpec((8, f), lambda i: (0, 0))],
        out_shape=[jax.ShapeDtypeStruct((n_pad, f), F32),
                   jax.ShapeDtypeStruct((8, f), F32)],
    )


@functools.lru_cache(maxsize=None)
def _msg_mm_builder(n_pad, d, f, chunks, use_counts, act):
    br = _row_block(n_pad, d)
    grid = n_pad // br
    nchunks = len(chunks)

    def body(*refs):
        x_ref = refs[0]
        p_refs = refs[1:1 + nchunks]
        rest = refs[1 + nchunks:]
        if use_counts:
            c_ref, w_ref, o_ref = rest
        else:
            w_ref, o_ref = rest
        m = jnp.concatenate([p[0] + p[1] for p in p_refs], axis=-1) \
            if nchunks > 1 else (p_refs[0][0] + p_refs[0][1])
        if use_counts:
            cnt = c_ref[0, :, 0:1] + c_ref[1, :, 0:1]
            m = m / (cnt + 1e-6)
        x = x_ref[...]
        y = jnp.dot(x + m, w_ref[...], preferred_element_type=F32)
        if act == "relu":
            o_ref[...] = jnp.maximum(y, 0.0)
        else:
            o_ref[...] = x * jax.nn.sigmoid(y)

    in_specs = [pl.BlockSpec((br, d), lambda i: (i, 0))]
    for dc in chunks:
        in_specs.append(pl.BlockSpec((2, br, dc), lambda i: (0, i, 0)))
    if use_counts:
        in_specs.append(pl.BlockSpec((2, br, 16), lambda i: (0, i, 0)))
    in_specs.append(pl.BlockSpec((d, f), lambda i: (0, 0)))
    return pl.pallas_call(
        body,
        grid=(grid,),
        in_specs=in_specs,
        out_specs=pl.BlockSpec((br, f), lambda i: (i, 0)),
        out_shape=jax.ShapeDtypeStruct((n_pad, f), F32),
    )


@functools.lru_cache(maxsize=None)
def _bn_relu_builder(n_pad, n_true, f):
    br = _row_block(n_pad, f)
    grid = n_pad // br
    inv_n = 1.0 / n_true

    def body(y_ref, st_ref, o_ref):
        mu = st_ref[0:1, :] * inv_n
        var = st_ref[1:2, :] * inv_n - mu * mu
        y = jnp.maximum((y_ref[...] - mu) * lax.rsqrt(var + 1e-5), 0.0)
        row = pl.program_id(0) * br + lax.broadcasted_iota(I32, (br, 1), 0)
        o_ref[...] = jnp.where(row < n_true, y, 0.0)

    return pl.pallas_call(
        body,
        grid=(grid,),
        in_specs=[pl.BlockSpec((br, f), lambda i: (i, 0)),
                  pl.BlockSpec((8, f), lambda i: (0, 0))],
        out_specs=pl.BlockSpec((br, f), lambda i: (i, 0)),
        out_shape=jax.ShapeDtypeStruct((n_pad, f), F32),
    )


@functools.lru_cache(maxsize=None)
def _mean_combine_builder(n_pad, dc):
    br = 2048 if n_pad % 2048 == 0 else n_pad
    grid = n_pad // br

    def body(p_ref, c_ref, o_ref):
        cnt = c_ref[0, :, 0:1] + c_ref[1, :, 0:1]
        o_ref[...] = (p_ref[0] + p_ref[1]) / jnp.maximum(cnt, 1.0)

    return pl.pallas_call(
        body,
        grid=(grid,),
        in_specs=[pl.BlockSpec((2, br, dc), lambda i: (0, i, 0)),
                  pl.BlockSpec((2, br, 16), lambda i: (0, i, 0))],
        out_specs=pl.BlockSpec((br, dc), lambda i: (i, 0)),
        out_shape=jax.ShapeDtypeStruct((n_pad, dc), F32),
    )


@functools.lru_cache(maxsize=None)
def _head_builder(k, f):
    def body(x_ref, w_ref, b_ref, o_ref):
        o_ref[...] = (jnp.dot(x_ref[...], w_ref[...], preferred_element_type=F32)
                      + b_ref[0:1, :])

    return pl.pallas_call(
        body,
        grid=(1,),
        in_specs=[pl.BlockSpec((64, k), lambda i: (0, 0)),
                  pl.BlockSpec((k, f), lambda i: (0, 0)),
                  pl.BlockSpec((8, f), lambda i: (0, 0))],
        out_specs=pl.BlockSpec((64, f), lambda i: (0, 0)),
        out_shape=jax.ShapeDtypeStruct((64, f), F32),
    )


def _bn_relu(y, st, n_true):
    return _bn_relu_builder(y.shape[0], n_true, y.shape[1])(y, st)


def _messages(x_table, inc, n_out):
    src, dst = inc[0], inc[1]
    e = src.shape[0]
    ar = jnp.arange(e, dtype=I32)
    ones = jnp.ones((e,), F32)
    gidx = jnp.concatenate([ar, ar])
    dsts = jnp.concatenate([src, dst])
    w = jnp.concatenate([-ones, ones])
    return _sc_scatter(x_table, gidx, dsts, w, n_out)


def _gather_diff(x_table, inc, n_out):
    src, dst = inc[0], inc[1]
    e = src.shape[0]
    ar = jnp.arange(e, dtype=I32)
    ones = jnp.ones((e,), F32)
    gidx = jnp.concatenate([dst, src])
    dsts = jnp.concatenate([ar, ar])
    w = jnp.concatenate([ones, -ones])
    return _sc_scatter(x_table, gidx, dsts, w, n_out)


def _msg_mm(x, parts, chunks, counts, w, act):
    n_pad, d = x.shape
    fn = _msg_mm_builder(n_pad, d, w.shape[1], chunks, counts is not None, act)
    args = [x] + list(parts)
    if counts is not None:
        args.append(counts)
    args.append(w)
    return fn(*args)


def _laguerre_bn(x, ei, ew, w0, w1, n_true):
    n_pad, d = x.shape
    parts, chunks = _sc_scatter(x, ei[0], ei[1], ew, n_true)
    y, st = _laguerre_builder(n_pad, n_true, d, w0.shape[1], chunks)(
        x, *parts, w0, w1)
    return _bn_relu(y, st, n_true)


def _scatter_mean(x_table, n_rows, idx, counts, n_out):
    ar = jnp.arange(n_rows, dtype=I32)
    parts, chunks = _sc_scatter(x_table, ar, idx, None, n_out)
    n_pad = parts[0].shape[1]
    outs = [_mean_combine_builder(n_pad, dc)(p, counts)
            for p, dc in zip(parts, chunks)]
    return jnp.concatenate(outs, axis=-1) if len(outs) > 1 else outs[0]


def kernel(x_t, x_s, edge_weight_t, edge_weight_s, edge_weight_t1,
           edge_weight_s1, params, edge_index_t, edge_index_s, edge_index,
           edge_index_t1, edge_index_s1, edge_index1, pos_t, pos_s,
           n_batch1, s_batch1):
    p = params
    filters = [64, 128, 256, 512]
    channels = [2, 2, 2, 2]
    n0 = x_t.shape[0]
    e0 = x_s.shape[0]
    n1 = edge_index_t1.shape[1] // 3 * 0 + 2000
    e1 = 2000
    ngraph = 64
    n0_pad = _ceil_to(n0 + 1, _ROW_PAD)
    e0_pad = _ceil_to(e0 + 1, _ROW_PAD)

    kt = _ceil_to(x_t.shape[1], 128)
    ks = _ceil_to(x_s.shape[1], 128)
    xtp = jnp.pad(x_t, ((0, n0_pad - n0), (0, kt - x_t.shape[1])))
    xsp = jnp.pad(x_s, ((0, e0_pad - e0), (0, ks - x_s.shape[1])))
    wt0 = jnp.pad(p["init_Wt"], ((0, kt - p["init_Wt"].shape[0]), (0, 0)))
    ws0 = jnp.pad(p["init_Ws"], ((0, ks - p["init_Ws"].shape[0]), (0, 0)))
    y, st = _mm_stats_builder(n0_pad, n0, kt, 64)(xtp, wt0)
    xt = _bn_relu(y, st, n0)
    y, st = _mm_stats_builder(e0_pad, e0, ks, 64)(xsp, ws0)
    xs = _bn_relu(y, st, e0)

    xt0, xs0 = xt, xs
    ei_t, ew_t = edge_index_t, edge_weight_t
    ei_s, ew_s = edge_index_s, edge_weight_s
    inc = edge_index
    nt, ne = n0, e0
    deg = _sc_bincount(inc.reshape(-1), nt)

    for i, f in enumerate(filters):
        for j in range(channels[i]):
            mt_parts, mt_chunks = _messages(xs0, inc, nt)
            ms_parts, ms_chunks = _gather_diff(xt0, inc, ne)
            xt_i = _msg_mm(xt0, mt_parts, mt_chunks, deg,
                           p["int%d%d_Wt" % (i, j)], "relu")
            xs_i = _msg_mm(xs0, ms_parts, ms_chunks, None,
                           p["int%d%d_Ws" % (i, j)], "relu")
            xt = _laguerre_bn(xt_i, ei_t, ew_t,
                              p["convt%d%d_W0" % (i, j)],
                              p["convt%d%d_W1" % (i, j)], nt)
            xs = _laguerre_bn(xs_i, ei_s, ew_s,
                              p["convs%d%d_W0" % (i, j)],
                              p["convs%d%d_W1" % (i, j)], ne)
            xt0 = jnp.concatenate([xt0, xt], axis=-1)
            xs0 = jnp.concatenate([xs0, xs], axis=-1)
        if i == 0:
            mt_parts, mt_chunks = _messages(xs0, inc, nt)
            ms_parts, ms_chunks = _gather_diff(xt0, inc, ne)
            at = _msg_mm(xt0, mt_parts, mt_chunks, deg, p["att_Wt"], "attsig")
            as_ = _msg_mm(xs0, ms_parts, ms_chunks, None, p["att_Ws"], "attsig")
            cnt_t = _sc_bincount(pos_t, n1)
            cnt_s = _sc_bincount(pos_s, e1)
            xt0 = _scatter_mean(at, nt, pos_t, cnt_t, n1)
            xs0 = _scatter_mean(as_, ne, pos_s, cnt_s, e1)
            ei_t, ew_t = edge_index_t1, edge_weight_t1
            ei_s, ew_s = edge_index_s1, edge_weight_s1
            inc = edge_index1
            nt, ne = n1, e1
            deg = _sc_bincount(inc.reshape(-1), nt)

    cnt_nb = _sc_bincount(n_batch1, ngraph)
    cnt_sb = _sc_bincount(s_batch1, ngraph)
    g_s = _scatter_mean(xs, ne, s_batch1, cnt_sb, ngraph)
    g_t = _scatter_mean(xt, nt, n_batch1, cnt_nb, ngraph)
    xg = jnp.concatenate([g_s, g_t], axis=-1)

    wout = p["out_W"]
    bout = jnp.broadcast_to(p["out_b"][None, :], (8, wout.shape[1]))
    return _head_builder(wout.shape[0], wout.shape[1])(xg, wout, bout)


# speedup vs baseline: 2.0536x; 1.0787x over previous
import functools

import jax
import jax.numpy as jnp
from jax import lax
from jax.experimental import pallas as pl
from jax.experimental.pallas import tpu as pltpu
from jax.experimental.pallas import tpu_sc as plsc

F32 = jnp.float32
I32 = jnp.int32

_CH = 64
_NW = 32
_STRIDE = _NW * _CH
_ROW_PAD = 1024
_SPMEM_BUDGET = 4 * 1024 * 1024


def _ceil_to(x, m):
    return -(-x // m) * m


def _feat_chunks(d, n_pad):
    max_dc = min(512, (_SPMEM_BUDGET // (4 * n_pad)) // 16 * 16)
    nc = -(-d // max_dc)
    base = (d // nc) // 16 * 16
    sizes = [base] * (nc - 1) + [d - base * (nc - 1)]
    assert all(16 <= s <= max_dc and s % 16 == 0 for s in sizes), (d, n_pad, sizes)
    return tuple(sizes)


@functools.lru_cache(maxsize=None)
def _sc_scatter_builder(nx, dc, nnz_pad, n_pad, mode):
    nchunk = nnz_pad // (_NW * _CH)
    rows_per_tile = n_pad // 16
    n_copies = rows_per_tile // _CH
    ncol = dc // 16
    mesh = plsc.VectorSubcoreMesh(core_axis_name="c", subcore_axis_name="s")

    scratch = []
    if mode != "ones":
        scratch.append(pltpu.VMEM((_CH,), I32))
    scratch.append(pltpu.VMEM((_CH,), I32))
    if mode == "gs":
        scratch.append(pltpu.VMEM((_CH,), F32))
    scratch += [
        pltpu.VMEM((_CH, dc), F32),
        pltpu.VMEM_SHARED((n_pad, dc), F32),
        pltpu.SemaphoreType.DMA,
    ]

    def kern(*args):
        if mode == "gs":
            (x_hbm, gidx_hbm, dst_hbm, w_hbm, out_hbm,
             gidx_v, dst_v, w_v, rows_v, acc, sem) = args
        elif mode == "g":
            (x_hbm, gidx_hbm, dst_hbm, out_hbm,
             gidx_v, dst_v, rows_v, acc, sem) = args
        else:
            (dst_hbm, out_hbm, dst_v, rows_v, acc, sem) = args
        c = lax.axis_index("c")
        s = lax.axis_index("s")
        wid = s * 2 + c
        t0 = s * rows_per_tile

        def fill(val):
            vec = jnp.full((16,), val, F32)

            def row(r, carry):
                for k in range(ncol):
                    rows_v[r, pl.ds(16 * k, 16)] = vec
                return carry

            lax.fori_loop(0, _CH, row, 0)

        fill(0.0)
        for j in range(n_copies):
            pltpu.sync_copy(rows_v, acc.at[pl.ds(t0 + j * _CH, _CH)])
        plsc.subcore_barrier()
        if mode == "ones":
            fill(1.0)

        def chunk(ci, carry):
            base = (wid * nchunk + ci) * _CH
            pltpu.sync_copy(dst_hbm.at[pl.ds(base, _CH)], dst_v)
            if mode != "ones":
                pltpu.sync_copy(gidx_hbm.at[pl.ds(base, _CH)], gidx_v)
                pltpu.async_copy(x_hbm.at[gidx_v], rows_v, sem).wait()
            if mode == "gs":
                pltpu.sync_copy(w_hbm.at[pl.ds(base, _CH)], w_v)

                def sgrp(g, carry2):
                    wg = w_v[pl.ds(g * 16, 16)]
                    for r16 in range(16):
                        ws = wg.at[jnp.full((16,), r16, I32)].get(
                            mode="promise_in_bounds")
                        for k in range(ncol):
                            sl = pl.ds(16 * k, 16)
                            rows_v[g * 16 + r16, sl] = rows_v[g * 16 + r16, sl] * ws
                    return carry2

                lax.fori_loop(0, _CH // 16, sgrp, 0)
            pltpu.sync_copy(rows_v, acc.at[dst_v], add=True)
            return carry

        lax.fori_loop(0, nchunk, chunk, 0)
        plsc.subcore_barrier()
        for j in range(n_copies):
            sl = pl.ds(t0 + j * _CH, _CH)
            pltpu.sync_copy(acc.at[sl], out_hbm.at[c, sl])

    return pl.kernel(
        kern,
        out_type=jax.ShapeDtypeStruct((2, n_pad, dc), F32),
        mesh=mesh,
        scratch_types=scratch,
        compiler_params=pltpu.CompilerParams(use_tc_tiling_on_sc=False),
    )


@functools.lru_cache(maxsize=None)
def _sc_diff_builder(nx, dc, e_pad):
    nchunk = e_pad // (_NW * _CH)
    ncol = dc // 16
    mesh = plsc.VectorSubcoreMesh(core_axis_name="c", subcore_axis_name="s")

    def kern(x_hbm, src_hbm, dst_hbm, out_hbm, si_v, di_v, ra_v, rb_v,
             sem_a, sem_b):
        c = lax.axis_index("c")
        s = lax.axis_index("s")
        wid = s * 2 + c

        def chunk(ci, carry):
            base = (wid * nchunk + ci) * _CH
            pltpu.sync_copy(dst_hbm.at[pl.ds(base, _CH)], di_v)
            pltpu.sync_copy(src_hbm.at[pl.ds(base, _CH)], si_v)
            ca = pltpu.async_copy(x_hbm.at[di_v], ra_v, sem_a)
            cb = pltpu.async_copy(x_hbm.at[si_v], rb_v, sem_b)
            ca.wait()
            cb.wait()

            def row(r, carry2):
                for k in range(ncol):
                    sl = pl.ds(16 * k, 16)
                    ra_v[r, sl] = ra_v[r, sl] - rb_v[r, sl]
                return carry2

            lax.fori_loop(0, _CH, row, 0)
            pltpu.sync_copy(ra_v, out_hbm.at[pl.ds(base, _CH)])
            return carry

        lax.fori_loop(0, nchunk, chunk, 0)

    return pl.kernel(
        kern,
        out_type=jax.ShapeDtypeStruct((e_pad, dc), F32),
        mesh=mesh,
        scratch_types=[
            pltpu.VMEM((_CH,), I32), pltpu.VMEM((_CH,), I32),
            pltpu.VMEM((_CH, dc), F32), pltpu.VMEM((_CH, dc), F32),
            pltpu.SemaphoreType.DMA, pltpu.SemaphoreType.DMA,
        ],
        compiler_params=pltpu.CompilerParams(use_tc_tiling_on_sc=False),
    )


def _pad_entries(arr, nnz_pad, value):
    n = arr.shape[0]
    if n == nnz_pad:
        return arr
    return jnp.concatenate([arr, jnp.full((nnz_pad - n,), value, arr.dtype)])


def _sc_scatter(x, gidx, dst, w, n_out):
    nnz = dst.shape[0]
    nnz_pad = _ceil_to(nnz, _STRIDE)
    n_pad = _ceil_to(n_out + 1, _ROW_PAD)
    gidx_p = _pad_entries(gidx, nnz_pad, 0)
    dst_p = _pad_entries(dst, nnz_pad, n_out)
    w_p = None if w is None else _pad_entries(w, nnz_pad, 0.0)
    d = x.shape[1]
    parts = []
    c0 = 0
    chunks = _feat_chunks(d, n_pad)
    for dc in chunks:
        xc = lax.slice_in_dim(x, c0, c0 + dc, axis=1)
        if w is None:
            fn = _sc_scatter_builder(x.shape[0], dc, nnz_pad, n_pad, "g")
            parts.append(fn(xc, gidx_p, dst_p))
        else:
            fn = _sc_scatter_builder(x.shape[0], dc, nnz_pad, n_pad, "gs")
            parts.append(fn(xc, gidx_p, dst_p, w_p))
        c0 += dc
    return parts, chunks


def _sc_bincount(idx, n_out):
    nnz = idx.shape[0]
    nnz_pad = _ceil_to(nnz, _STRIDE)
    n_pad = _ceil_to(n_out + 1, _ROW_PAD)
    dst_p = _pad_entries(idx, nnz_pad, n_out)
    fn = _sc_scatter_builder(0, 16, nnz_pad, n_pad, "ones")
    return fn(dst_p)


def _row_block(n_pad, d_tot):
    br = 2048 if n_pad % 2048 == 0 else n_pad
    if d_tot >= 704 and br > 1024:
        br = 1024
    return br


@functools.lru_cache(maxsize=None)
def _mm_stats_builder(n_pad, n_true, k, f):
    br = _row_block(n_pad, k)
    grid = n_pad // br

    def body(x_ref, w_ref, y_ref, st_ref):
        y = jnp.dot(x_ref[...], w_ref[...], preferred_element_type=F32)
        y_ref[...] = y
        i = pl.program_id(0)

        @pl.when(i == 0)
        def _():
            st_ref[...] = jnp.zeros_like(st_ref)

        row = i * br + lax.broadcasted_iota(I32, (br, 1), 0)
        ym = jnp.where(row < n_true, y, 0.0)
        st_ref[0:1, :] = st_ref[0:1, :] + jnp.sum(ym, axis=0, keepdims=True)
        st_ref[1:2, :] = st_ref[1:2, :] + jnp.sum(ym * ym, axis=0, keepdims=True)

    return pl.pallas_call(
        body,
        grid=(grid,),
        in_specs=[pl.BlockSpec((br, k), lambda i: (i, 0)),
                  pl.BlockSpec((k, f), lambda i: (0, 0))],
        out_specs=[pl.BlockSpec((br, f), lambda i: (i, 0)),
                   pl.BlockSpec((8, f), lambda i: (0, 0))],
        out_shape=[jax.ShapeDtypeStruct((n_pad, f), F32),
                   jax.ShapeDtypeStruct((8, f), F32)],
    )


@functools.lru_cache(maxsize=None)
def _laguerre_builder(n_pad, n_true, d, f, chunks):
    br = _row_block(n_pad, d)
    grid = n_pad // br
    nchunks = len(chunks)

    def body(*refs):
        x_ref = refs[0]
        p_refs = refs[1:1 + nchunks]
        w0_ref, w1_ref, y_ref, st_ref = refs[1 + nchunks:]
        x = x_ref[...]
        lx = jnp.concatenate([p[0] + p[1] for p in p_refs], axis=-1) \
            if nchunks > 1 else (p_refs[0][0] + p_refs[0][1])
        y = (jnp.dot(x, w0_ref[...], preferred_element_type=F32)
             + jnp.dot(x - lx, w1_ref[...], preferred_element_type=F32))
        y_ref[...] = y
        i = pl.program_id(0)

        @pl.when(i == 0)
        def _():
            st_ref[...] = jnp.zeros_like(st_ref)

        row = i * br + lax.broadcasted_iota(I32, (br, 1), 0)
        ym = jnp.where(row < n_true, y, 0.0)
        st_ref[0:1, :] = st_ref[0:1, :] + jnp.sum(ym, axis=0, keepdims=True)
        st_ref[1:2, :] = st_ref[1:2, :] + jnp.sum(ym * ym, axis=0, keepdims=True)

    in_specs = [pl.BlockSpec((br, d), lambda i: (i, 0))]
    for dc in chunks:
        in_specs.append(pl.BlockSpec((2, br, dc), lambda i: (0, i, 0)))
    in_specs += [pl.BlockSpec((d, f), lambda i: (0, 0)),
                 pl.BlockSpec((d, f), lambda i: (0, 0))]
    return pl.pallas_call(
        body,
        grid=(grid,),
        in_specs=in_specs,
        out_specs=[pl.BlockSpec((br, f), lambda i: (i, 0)),
                   pl.BlockSpec((8, f), lambda i: (0, 0))],
        out_shape=[jax.ShapeDtypeStruct((n_pad, f), F32),
                   jax.ShapeDtypeStruct((8, f), F32)],
    )


@functools.lru_cache(maxsize=None)
def _msg_mm_builder(n_pad, d, f, chunks, use_counts, act):
    br = _row_block(n_pad, d)
    grid = n_pad // br
    nchunks = 1 if chunks is None else len(chunks)

    def body(*refs):
        x_ref = refs[0]
        p_refs = refs[1:1 + nchunks]
        rest = refs[1 + nchunks:]
        if use_counts:
            c_ref, w_ref, o_ref = rest
        else:
            w_ref, o_ref = rest
        if chunks is None:
            m = p_refs[0][...]
        else:
            m = jnp.concatenate([p[0] + p[1] for p in p_refs], axis=-1) \
                if nchunks > 1 else (p_refs[0][0] + p_refs[0][1])
        if use_counts:
            cnt = c_ref[0, :, 0:1] + c_ref[1, :, 0:1]
            m = m / (cnt + 1e-6)
        x = x_ref[...]
        y = jnp.dot(x + m, w_ref[...], preferred_element_type=F32)
        if act == "relu":
            o_ref[...] = jnp.maximum(y, 0.0)
        else:
            o_ref[...] = x * jax.nn.sigmoid(y)

    in_specs = [pl.BlockSpec((br, d), lambda i: (i, 0))]
    if chunks is None:
        in_specs.append(pl.BlockSpec((br, d), lambda i: (i, 0)))
    else:
        for dc in chunks:
            in_specs.append(pl.BlockSpec((2, br, dc), lambda i: (0, i, 0)))
    if use_counts:
        in_specs.append(pl.BlockSpec((2, br, 16), lambda i: (0, i, 0)))
    in_specs.append(pl.BlockSpec((d, f), lambda i: (0, 0)))
    return pl.pallas_call(
        body,
        grid=(grid,),
        in_specs=in_specs,
        out_specs=pl.BlockSpec((br, f), lambda i: (i, 0)),
        out_shape=jax.ShapeDtypeStruct((n_pad, f), F32),
    )


@functools.lru_cache(maxsize=None)
def _bn_relu_builder(n_pad, n_true, f):
    br = _row_block(n_pad, f)
    grid = n_pad // br
    inv_n = 1.0 / n_true

    def body(y_ref, st_ref, o_ref):
        mu = st_ref[0:1, :] * inv_n
        var = st_ref[1:2, :] * inv_n - mu * mu
        y = jnp.maximum((y_ref[...] - mu) * lax.rsqrt(var + 1e-5), 0.0)
        row = pl.program_id(0) * br + lax.broadcasted_iota(I32, (br, 1), 0)
        o_ref[...] = jnp.where(row < n_true, y, 0.0)

    return pl.pallas_call(
        body,
        grid=(grid,),
        in_specs=[pl.BlockSpec((br, f), lambda i: (i, 0)),
                  pl.BlockSpec((8, f), lambda i: (0, 0))],
        out_specs=pl.BlockSpec((br, f), lambda i: (i, 0)),
        out_shape=jax.ShapeDtypeStruct((n_pad, f), F32),
    )


@functools.lru_cache(maxsize=None)
def _mean_combine_builder(n_pad, dc):
    br = 2048 if n_pad % 2048 == 0 else n_pad
    grid = n_pad // br

    def body(p_ref, c_ref, o_ref):
        cnt = c_ref[0, :, 0:1] + c_ref[1, :, 0:1]
        o_ref[...] = (p_ref[0] + p_ref[1]) / jnp.maximum(cnt, 1.0)

    return pl.pallas_call(
        body,
        grid=(grid,),
        in_specs=[pl.BlockSpec((2, br, dc), lambda i: (0, i, 0)),
                  pl.BlockSpec((2, br, 16), lambda i: (0, i, 0))],
        out_specs=pl.BlockSpec((br, dc), lambda i: (i, 0)),
        out_shape=jax.ShapeDtypeStruct((n_pad, dc), F32),
    )


@functools.lru_cache(maxsize=None)
def _head_builder(k, f):
    def body(x_ref, w_ref, b_ref, o_ref):
        o_ref[...] = (jnp.dot(x_ref[...], w_ref[...], preferred_element_type=F32)
                      + b_ref[0:1, :])

    return pl.pallas_call(
        body,
        grid=(1,),
        in_specs=[pl.BlockSpec((64, k), lambda i: (0, 0)),
                  pl.BlockSpec((k, f), lambda i: (0, 0)),
                  pl.BlockSpec((8, f), lambda i: (0, 0))],
        out_specs=pl.BlockSpec((64, f), lambda i: (0, 0)),
        out_shape=jax.ShapeDtypeStruct((64, f), F32),
    )


def _bn_relu(y, st, n_true):
    return _bn_relu_builder(y.shape[0], n_true, y.shape[1])(y, st)


def _messages(x_table, inc, n_out):
    src, dst = inc[0], inc[1]
    e = src.shape[0]
    ar = jnp.arange(e, dtype=I32)
    ones = jnp.ones((e,), F32)
    gidx = jnp.concatenate([ar, ar])
    dsts = jnp.concatenate([src, dst])
    w = jnp.concatenate([-ones, ones])
    return _sc_scatter(x_table, gidx, dsts, w, n_out)


def _gather_diff(x_table, inc, n_out):
    src, dst = inc[0], inc[1]
    e = src.shape[0]
    e_pad = _ceil_to(e, _STRIDE)
    src_p = _pad_entries(src, e_pad, 0)
    dst_p = _pad_entries(dst, e_pad, 0)
    d = x_table.shape[1]
    nc = -(-d // 512)
    base = (d // nc) // 16 * 16
    sizes = [base] * (nc - 1) + [d - base * (nc - 1)]
    outs = []
    c0 = 0
    for dc in sizes:
        xc = lax.slice_in_dim(x_table, c0, c0 + dc, axis=1)
        outs.append(_sc_diff_builder(x_table.shape[0], dc, e_pad)(
            xc, src_p, dst_p))
        c0 += dc
    return jnp.concatenate(outs, axis=-1) if len(outs) > 1 else outs[0]


def _msg_mm(x, parts, chunks, counts, w, act):
    n_pad, d = x.shape
    fn = _msg_mm_builder(n_pad, d, w.shape[1], chunks, counts is not None, act)
    args = [x] + list(parts)
    if counts is not None:
        args.append(counts)
    args.append(w)
    return fn(*args)


def _laguerre_bn(x, ei, ew, w0, w1, n_true):
    n_pad, d = x.shape
    parts, chunks = _sc_scatter(x, ei[0], ei[1], ew, n_true)
    y, st = _laguerre_builder(n_pad, n_true, d, w0.shape[1], chunks)(
        x, *parts, w0, w1)
    return _bn_relu(y, st, n_true)


def _scatter_mean(x_table, n_rows, idx, counts, n_out):
    ar = jnp.arange(n_rows, dtype=I32)
    parts, chunks = _sc_scatter(x_table, ar, idx, None, n_out)
    n_pad = parts[0].shape[1]
    outs = [_mean_combine_builder(n_pad, dc)(p, counts)
            for p, dc in zip(parts, chunks)]
    return jnp.concatenate(outs, axis=-1) if len(outs) > 1 else outs[0]


def kernel(x_t, x_s, edge_weight_t, edge_weight_s, edge_weight_t1,
           edge_weight_s1, params, edge_index_t, edge_index_s, edge_index,
           edge_index_t1, edge_index_s1, edge_index1, pos_t, pos_s,
           n_batch1, s_batch1):
    p = params
    filters = [64, 128, 256, 512]
    channels = [2, 2, 2, 2]
    n0 = x_t.shape[0]
    e0 = x_s.shape[0]
    n1 = edge_index_t1.shape[1] // 3 * 0 + 2000
    e1 = 2000
    ngraph = 64
    n0_pad = _ceil_to(n0 + 1, _ROW_PAD)
    e0_pad = _ceil_to(e0 + 1, _ROW_PAD)

    kt = _ceil_to(x_t.shape[1], 128)
    ks = _ceil_to(x_s.shape[1], 128)
    xtp = jnp.pad(x_t, ((0, n0_pad - n0), (0, kt - x_t.shape[1])))
    xsp = jnp.pad(x_s, ((0, e0_pad - e0), (0, ks - x_s.shape[1])))
    wt0 = jnp.pad(p["init_Wt"], ((0, kt - p["init_Wt"].shape[0]), (0, 0)))
    ws0 = jnp.pad(p["init_Ws"], ((0, ks - p["init_Ws"].shape[0]), (0, 0)))
    y, st = _mm_stats_builder(n0_pad, n0, kt, 64)(xtp, wt0)
    xt = _bn_relu(y, st, n0)
    y, st = _mm_stats_builder(e0_pad, e0, ks, 64)(xsp, ws0)
    xs = _bn_relu(y, st, e0)

    xt0, xs0 = xt, xs
    ei_t, ew_t = edge_index_t, edge_weight_t
    ei_s, ew_s = edge_index_s, edge_weight_s
    inc = edge_index
    nt, ne = n0, e0
    deg = _sc_bincount(inc.reshape(-1), nt)

    for i, f in enumerate(filters):
        for j in range(channels[i]):
            mt_parts, mt_chunks = _messages(xs0, inc, nt)
            m_s = _gather_diff(xt0, inc, ne)
            xt_i = _msg_mm(xt0, mt_parts, mt_chunks, deg,
                           p["int%d%d_Wt" % (i, j)], "relu")
            xs_i = _msg_mm(xs0, [m_s], None, None,
                           p["int%d%d_Ws" % (i, j)], "relu")
            xt = _laguerre_bn(xt_i, ei_t, ew_t,
                              p["convt%d%d_W0" % (i, j)],
                              p["convt%d%d_W1" % (i, j)], nt)
            xs = _laguerre_bn(xs_i, ei_s, ew_s,
                              p["convs%d%d_W0" % (i, j)],
                              p["convs%d%d_W1" % (i, j)], ne)
            xt0 = jnp.concatenate([xt0, xt], axis=-1)
            xs0 = jnp.concatenate([xs0, xs], axis=-1)
        if i == 0:
            mt_parts, mt_chunks = _messages(xs0, inc, nt)
            m_s = _gather_diff(xt0, inc, ne)
            at = _msg_mm(xt0, mt_parts, mt_chunks, deg, p["att_Wt"], "attsig")
            as_ = _msg_mm(xs0, [m_s], None, None, p["att_Ws"], "attsig")
            cnt_t = _sc_bincount(pos_t, n1)
            cnt_s = _sc_bincount(pos_s, e1)
            xt0 = _scatter_mean(at, nt, pos_t, cnt_t, n1)
            xs0 = _scatter_mean(as_, ne, pos_s, cnt_s, e1)
            ei_t, ew_t = edge_index_t1, edge_weight_t1
            ei_s, ew_s = edge_index_s1, edge_weight_s1
            inc = edge_index1
            nt, ne = n1, e1
            deg = _sc_bincount(inc.reshape(-1), nt)

    cnt_nb = _sc_bincount(n_batch1, ngraph)
    cnt_sb = _sc_bincount(s_batch1, ngraph)
    g_s = _scatter_mean(xs, ne, s_batch1, cnt_sb, ngraph)
    g_t = _scatter_mean(xt, nt, n_batch1, cnt_nb, ngraph)
    xg = jnp.concatenate([g_s, g_t], axis=-1)

    wout = p["out_W"]
    bout = jnp.broadcast_to(p["out_b"][None, :], (8, wout.shape[1]))
    return _head_builder(wout.shape[0], wout.shape[1])(xg, wout, bout)
